# Initial kernel scaffold; baseline (speedup 1.0000x reference)
import jax
import jax.numpy as jnp
from jax.experimental import pallas as pl


def kernel(x, edge_index, edge_attr, iso_type_2, iso_type_3, assignment_index_2, assignment_index_3, edge_index_2, edge_index_3, batch, batch_2, batch_3, params):
    raise NotImplementedError("write your pallas kernel here")


# speedup vs baseline: 1.0992x; 1.0992x over previous
import functools

import jax
import jax.numpy as jnp
from jax import lax
from jax.experimental import pallas as pl
from jax.experimental.pallas import tpu as pltpu
from jax.experimental.pallas import tpu_sc as plsc

F32 = jnp.float32
NC, NS, LN = 2, 16, 16
NW = NC * NS
CH = 128
D = 128


def _rup(a, b):
    return -(-a // b) * b


def _pad_idx(idx, ep, fill):
    idx = idx.astype(jnp.int32)
    e = idx.shape[0]
    if ep > e:
        idx = jnp.concatenate([idx, jnp.full((ep - e,), fill, jnp.int32)])
    return idx.reshape(NW, ep // CH // NW, CH)


def _acc_rows(n):
    return NS * _rup(_rup(n + 1, NS) // NS, CH)


_MESH = dict(core_axis_name="c", subcore_axis_name="s")


def _sc_gather(table, idx2d, ep):
    k = ep // CH // NW

    @functools.partial(
        pl.kernel,
        out_type=jax.ShapeDtypeStruct((ep, D), F32),
        mesh=plsc.VectorSubcoreMesh(**_MESH),
        scratch_types=[
            pltpu.VMEM((k, CH), jnp.int32),
            pltpu.VMEM((CH, D), F32),
            pltpu.SemaphoreType.DMA,
        ],
    )
    def kern(table_h, idx_h, out_h, idx_v, rows_v, sem):
        wid = lax.axis_index("s") * NC + lax.axis_index("c")
        base = wid * k
        pltpu.sync_copy(idx_h.at[wid], idx_v)

        def body(j, _):
            pltpu.async_copy(table_h.at[idx_v.at[j]], rows_v, sem).wait()
            pltpu.sync_copy(rows_v, out_h.at[pl.ds((base + j) * CH, CH)])
            return 0

        lax.fori_loop(0, k, body, 0)

    return kern(table, idx2d)


def _zero_fill(buf, rows):
    def zb(r, _):
        for c in range(D // LN):
            buf[r, pl.ds(c * LN, LN)] = jnp.zeros((LN,), F32)
        return 0

    lax.fori_loop(0, rows, zb, 0)


def _sc_scatter_add(data, idx2d, ep, npr):
    k = ep // CH // NW
    rt = npr // NS

    @functools.partial(
        pl.kernel,
        out_type=jax.ShapeDtypeStruct((2, npr, D), F32),
        mesh=plsc.VectorSubcoreMesh(**_MESH),
        scratch_types=[
            pltpu.VMEM((k, CH), jnp.int32),
            pltpu.VMEM((CH, D), F32),
            pltpu.VMEM_SHARED((npr, D), F32),
            pltpu.SemaphoreType.DMA,
        ],
    )
    def kern(data_h, idx_h, out_h, idx_v, rows_v, acc_s, sem):
        cid = lax.axis_index("c")
        sid = lax.axis_index("s")
        wid = sid * NC + cid
        base = wid * k
        pltpu.sync_copy(idx_h.at[wid], idx_v)
        _zero_fill(rows_v, CH)

        def zs(r, _):
            pltpu.sync_copy(rows_v, acc_s.at[pl.ds(sid * rt + r * CH, CH)])
            return 0

        lax.fori_loop(0, rt // CH, zs, 0)
        plsc.subcore_barrier()

        def body(j, _):
            pltpu.async_copy(data_h.at[pl.ds((base + j) * CH, CH)], rows_v,
                             sem).wait()
            pltpu.sync_copy(rows_v, acc_s.at[idx_v.at[j]], add=True)
            return 0

        lax.fori_loop(0, k, body, 0)
        plsc.subcore_barrier()

        def rd(r, _):
            pltpu.sync_copy(acc_s.at[pl.ds(sid * rt + r * CH, CH)], rows_v)
            pltpu.sync_copy(rows_v, out_h.at[cid, pl.ds(sid * rt + r * CH, CH)])
            return 0

        lax.fori_loop(0, rt // CH, rd, 0)

    return kern(data, idx2d)


def _sc_gather_scatter(table, src2d, dst2d, ep, npr):
    k = ep // CH // NW
    rt = npr // NS

    @functools.partial(
        pl.kernel,
        out_type=jax.ShapeDtypeStruct((2, npr, D), F32),
        mesh=plsc.VectorSubcoreMesh(**_MESH),
        scratch_types=[
            pltpu.VMEM((k, CH), jnp.int32),
            pltpu.VMEM((k, CH), jnp.int32),
            pltpu.VMEM((CH, D), F32),
            pltpu.VMEM_SHARED((npr, D), F32),
            pltpu.SemaphoreType.DMA,
        ],
    )
    def kern(table_h, src_h, dst_h, out_h, sidx_v, didx_v, rows_v, acc_s, sem):
        cid = lax.axis_index("c")
        sid = lax.axis_index("s")
        wid = sid * NC + cid
        base = wid * k
        pltpu.sync_copy(src_h.at[wid], sidx_v)
        pltpu.sync_copy(dst_h.at[wid], didx_v)
        _zero_fill(rows_v, CH)

        def zs(r, _):
            pltpu.sync_copy(rows_v, acc_s.at[pl.ds(sid * rt + r * CH, CH)])
            return 0

        lax.fori_loop(0, rt // CH, zs, 0)
        plsc.subcore_barrier()

        def body(j, _):
            pltpu.async_copy(table_h.at[sidx_v.at[j]], rows_v, sem).wait()
            pltpu.sync_copy(rows_v, acc_s.at[didx_v.at[j]], add=True)
            return 0

        lax.fori_loop(0, k, body, 0)
        plsc.subcore_barrier()

        def rd(r, _):
            pltpu.sync_copy(acc_s.at[pl.ds(sid * rt + r * CH, CH)], rows_v)
            pltpu.sync_copy(rows_v, out_h.at[cid, pl.ds(sid * rt + r * CH, CH)])
            return 0

        lax.fori_loop(0, rt // CH, rd, 0)

    return kern(table, src2d, dst2d)


def _elu(v):
    return jnp.where(v > 0, v, jnp.exp(jnp.minimum(v, 0.0)) - 1.0)


def _ones_col(blkn):
    return (lax.broadcasted_iota(jnp.int32, (blkn, 64), 1) == 0).astype(F32)


def _tc_nnconv_edge(ea_p, xg, w1t, b1, w2t, b2, mi, mo, blk):
    ep, eaw = ea_p.shape
    kw = w2t.shape[1]

    def body(ea_ref, xg_ref, w1_ref, b1_ref, w2_ref, b2_ref, out_ref):
        h = jnp.maximum(
            jnp.dot(ea_ref[...], w1_ref[...], preferred_element_type=F32)
            + b1_ref[...], 0.0)
        theta = jnp.dot(h.astype(jnp.bfloat16), w2_ref[...],
                        preferred_element_type=F32) + b2_ref[...]
        xb = xg_ref[...]
        acc = xb[:, 0:1] * theta[:, 0:mo]
        for i in range(1, mi):
            acc = acc + xb[:, i:i + 1] * theta[:, i * mo:(i + 1) * mo]
        out_ref[...] = jnp.concatenate(
            [acc, jnp.zeros((blk, D - mo), F32)], axis=1)

    return pl.pallas_call(
        body,
        grid=(ep // blk,),
        in_specs=[
            pl.BlockSpec((blk, eaw), lambda i: (i, 0)),
            pl.BlockSpec((blk, D), lambda i: (i, 0)),
            pl.BlockSpec((eaw, 128), lambda i: (0, 0)),
            pl.BlockSpec((1, 128), lambda i: (0, 0)),
            pl.BlockSpec((128, kw), lambda i: (0, 0)),
            pl.BlockSpec((1, kw), lambda i: (0, 0)),
        ],
        out_specs=pl.BlockSpec((blk, D), lambda i: (i, 0)),
        out_shape=jax.ShapeDtypeStruct((ep, D), F32),
    )(ea_p, xg, w1t, b1, w2t, b2)


def _tc_nnconv_node(agg2, x_prev, xcols, root, bias, blkn=1000):
    n = x_prev.shape[0]
    mo = root.shape[1]

    def body(a_ref, x_ref, r_ref, b_ref, out_ref):
        v = (a_ref[0, :, :mo] + a_ref[1, :, :mo]
             + jnp.dot(x_ref[...][:, :xcols], r_ref[...],
                       preferred_element_type=F32)
             + b_ref[...])
        feat = _elu(v)
        if mo < 64:
            feat = jnp.concatenate([feat, jnp.zeros((blkn, 64 - mo), F32)],
                                   axis=1)
        out_ref[...] = jnp.concatenate([feat, _ones_col(blkn)], axis=1)

    return pl.pallas_call(
        body,
        grid=(n // blkn,),
        in_specs=[
            pl.BlockSpec((2, blkn, D), lambda i: (0, i, 0)),
            pl.BlockSpec((blkn, x_prev.shape[1]), lambda i: (i, 0)),
            pl.BlockSpec(root.shape, lambda i: (0, 0)),
            pl.BlockSpec((1, mo), lambda i: (0, 0)),
        ],
        out_specs=pl.BlockSpec((blkn, D), lambda i: (i, 0)),
        out_shape=jax.ShapeDtypeStruct((n, D), F32),
    )(agg2, x_prev, root, bias)


def _tc_pool_concat(sum2, iso_col, n, noh, blkn=1000):
    def body(s_ref, i_ref, out_ref):
        s = s_ref[0] + s_ref[1]
        mean = s[:, :64] / jnp.maximum(s[:, 64:65], 1.0)
        oh = (i_ref[...] == lax.broadcasted_iota(jnp.int32, (blkn, noh),
                                                 1)).astype(F32)
        out_ref[...] = jnp.concatenate([mean, oh], axis=1)

    return pl.pallas_call(
        body,
        grid=(n // blkn,),
        in_specs=[
            pl.BlockSpec((2, blkn, D), lambda i: (0, i, 0)),
            pl.BlockSpec((blkn, 1), lambda i: (i, 0)),
        ],
        out_specs=pl.BlockSpec((blkn, 64 + noh), lambda i: (i, 0)),
        out_shape=jax.ShapeDtypeStruct((n, 64 + noh), F32),
    )(sum2, iso_col)


def _tc_graphconv_node(agg2, h_prev, din, wrel_t, brel, wroot_t, blkn=1000):
    n = h_prev.shape[0]
    mo = wrel_t.shape[1]

    def body(a_ref, h_ref, wr_ref, br_ref, wo_ref, out_ref):
        v = (jnp.dot(a_ref[0, :, :din] + a_ref[1, :, :din], wr_ref[...],
                     preferred_element_type=F32) + br_ref[...]
             + jnp.dot(h_ref[...][:, :din], wo_ref[...],
                       preferred_element_type=F32))
        out_ref[...] = jnp.concatenate([_elu(v), _ones_col(blkn)], axis=1)

    return pl.pallas_call(
        body,
        grid=(n // blkn,),
        in_specs=[
            pl.BlockSpec((2, blkn, D), lambda i: (0, i, 0)),
            pl.BlockSpec((blkn, h_prev.shape[1]), lambda i: (i, 0)),
            pl.BlockSpec((din, mo), lambda i: (0, 0)),
            pl.BlockSpec((1, mo), lambda i: (0, 0)),
            pl.BlockSpec((din, mo), lambda i: (0, 0)),
        ],
        out_specs=pl.BlockSpec((blkn, D), lambda i: (i, 0)),
        out_shape=jax.ShapeDtypeStruct((n, D), F32),
    )(agg2, h_prev, wrel_t, brel, wroot_t)


def _tc_head(h1, h2, h3, bt1, bt2, bt3, nb, fw1, fb1, fw2, fb2, fw3, fb3):
    def seg_mean(h, b, n):
        m = (lax.broadcasted_iota(jnp.int32, (nb, n), 0) == b).astype(F32)
        s = jnp.dot(m, h[:, :64], preferred_element_type=F32)
        c = jnp.sum(m, axis=1, keepdims=True)
        return s / jnp.maximum(c, 1.0)

    def body(h1_ref, h2_ref, h3_ref, b1_ref, b2_ref, b3_ref, w1_ref, fb1_ref,
             w2_ref, fb2_ref, w3_ref, fb3_ref, out_ref):
        x1 = seg_mean(h1_ref[...], b1_ref[...], h1_ref.shape[0])
        x2 = seg_mean(h2_ref[...], b2_ref[...], h2_ref.shape[0])
        x3 = seg_mean(h3_ref[...], b3_ref[...], h3_ref.shape[0])
        z = jnp.concatenate([x1, x2, x3], axis=1)
        z = _elu(jnp.dot(z, w1_ref[...], preferred_element_type=F32)
                 + fb1_ref[...])
        z = _elu(jnp.dot(z, w2_ref[...], preferred_element_type=F32)
                 + fb2_ref[...])
        out_ref[...] = (jnp.dot(z, w3_ref[...], preferred_element_type=F32)
                        + fb3_ref[...])

    return pl.pallas_call(
        body,
        out_shape=jax.ShapeDtypeStruct((nb, 1), F32),
    )(h1, h2, h3, bt1, bt2, bt3, fw1, fb1, fw2, fb2, fw3, fb3)


def _nnconv_weights(p, tag, mi, mo):
    w1t = p[f'{tag}_W1'].T
    ea_in = w1t.shape[0]
    eaw = _rup(ea_in, 8)
    w1t = jnp.pad(w1t, ((0, eaw - ea_in), (0, 0)))
    b1 = p[f'{tag}_b1'].reshape(1, 128)
    w2t = p[f'{tag}_W2'].T.reshape(128, mi * mo)
    b2 = p[f'{tag}_b2'].reshape(1, mi * mo)
    return w1t, b1, w2t.astype(jnp.bfloat16), b2, eaw


def kernel(x, edge_index, edge_attr, iso_type_2, iso_type_3,
           assignment_index_2, assignment_index_3, edge_index_2, edge_index_3,
           batch, batch_2, batch_3, params):
    p = params
    n, f_in = x.shape
    e = edge_index.shape[1]
    n2 = iso_type_2.shape[0]
    n3 = iso_type_3.shape[0]
    noh2 = p['conv4_Wrel'].shape[1] - 64
    noh3 = p['conv6_Wrel'].shape[1] - 64
    nb = 256

    ep = _rup(e, NW * CH)
    blk = next(b for b in (640, 512, 256, 128) if ep % b == 0)
    npr = _acc_rows(n)
    npr2 = _acc_rows(n2)
    npr3 = _acc_rows(n3)

    src = _pad_idx(edge_index[0], ep, 0)
    dst = _pad_idx(edge_index[1], ep, n)
    ea_p = jnp.pad(edge_attr, ((0, ep - e), (0, 3)))

    x_p = jnp.pad(x, ((0, 0), (0, D - f_in)))

    w1t, b1, w2t, b2, eaw = _nnconv_weights(p, 'nn1', f_in, 32)
    xg = _sc_gather(x_p, src, ep)
    msg = _tc_nnconv_edge(ea_p, xg, w1t, b1, w2t, b2, f_in, 32, blk)
    agg = _sc_scatter_add(msg, dst, ep, npr)
    h = _tc_nnconv_node(agg, x, f_in, p['conv1_root'],
                        p['conv1_bias'].reshape(1, -1))

    w1t, b1, w2t, b2, eaw = _nnconv_weights(p, 'nn2', 32, 64)
    xg = _sc_gather(h, src, ep)
    msg = _tc_nnconv_edge(ea_p, xg, w1t, b1, w2t, b2, 32, 64, blk)
    agg = _sc_scatter_add(msg, dst, ep, npr)
    h = _tc_nnconv_node(agg, h, 32, p['conv2_root'],
                        p['conv2_bias'].reshape(1, -1))

    w1t, b1, w2t, b2, eaw = _nnconv_weights(p, 'nn3', 64, 64)
    xg = _sc_gather(h, src, ep)
    msg = _tc_nnconv_edge(ea_p, xg, w1t, b1, w2t, b2, 64, 64, blk)
    agg = _sc_scatter_add(msg, dst, ep, npr)
    h = _tc_nnconv_node(agg, h, 64, p['conv3_root'],
                        p['conv3_bias'].reshape(1, -1))

    def branch(ai, ei, iso, n_lvl, npr_lvl, noh, ca, cb):
        ap = _rup(ai.shape[1], NW * CH)
        elp = _rup(ei.shape[1], NW * CH)
        s2 = _sc_gather_scatter(h, _pad_idx(ai[0], ap, 0),
                                _pad_idx(ai[1], ap, n_lvl), ap, npr_lvl)
        hf = _tc_pool_concat(s2, iso.reshape(n_lvl, 1).astype(jnp.int32),
                             n_lvl, noh)
        src_l = _pad_idx(ei[0], elp, 0)
        dst_l = _pad_idx(ei[1], elp, n_lvl)
        g = _sc_gather_scatter(hf, src_l, dst_l, elp, npr_lvl)
        ha = _tc_graphconv_node(g, hf, 64 + noh, p[f'{ca}_Wrel'].T,
                                p[f'{ca}_brel'].reshape(1, -1),
                                p[f'{ca}_Wroot'].T)
        g = _sc_gather_scatter(ha, src_l, dst_l, elp, npr_lvl)
        return _tc_graphconv_node(g, ha, 64, p[f'{cb}_Wrel'].T,
                                  p[f'{cb}_brel'].reshape(1, -1),
                                  p[f'{cb}_Wroot'].T)

    h2 = branch(assignment_index_2, edge_index_2, iso_type_2, n2, npr2, noh2,
                'conv4', 'conv5')
    h3 = branch(assignment_index_3, edge_index_3, iso_type_3, n3, npr3, noh3,
                'conv6', 'conv7')

    return _tc_head(h, h2, h3,
                    batch.reshape(1, n).astype(jnp.int32),
                    batch_2.reshape(1, n2).astype(jnp.int32),
                    batch_3.reshape(1, n3).astype(jnp.int32),
                    nb,
                    p['fc1_W'].T, p['fc1_b'].reshape(1, -1),
                    p['fc2_W'].T, p['fc2_b'].reshape(1, -1),
                    p['fc3_W'].T, p['fc3_b'].reshape(1, -1))
